# baseline (speedup 1.0000x reference)
import functools

import jax
import jax.numpy as jnp
import numpy as np
from jax import lax
from jax.experimental import pallas as pl
from jax.experimental.pallas import tpu as pltpu
from jax.experimental.pallas import tpu_sc as plsc

N_HEAD = 16
K_KEEP = 512
RECENCY = 64


def _qkv_kernel(x_ref, w_ref, o_ref, *, hpb, dh):
    r = jax.lax.dot_general(
        x_ref[...], w_ref[...],
        (((1,), (0,)), ((), ())),
        preferred_element_type=jnp.float32)
    r = r.astype(jnp.bfloat16)
    for ih in range(hpb):
        o_ref[ih, 0, :, :] = r[:, ih * dh:(ih + 1) * dh]


def _importance_kernel(q_ref, k_ref, imp_ref, thr_ref, colsum_ref, *,
                       t, bq, scale, k_keep, recency):
    qi = pl.program_id(2)

    @pl.when(qi == 0)
    def _():
        colsum_ref[...] = jnp.zeros_like(colsum_ref)

    q = q_ref[0, 0, :, :]
    k = k_ref[0, 0, :, :]
    s = jax.lax.dot_general(
        q, k, (((1,), (1,)), ((), ())),
        preferred_element_type=jnp.float32) * scale
    row = qi * bq + jax.lax.broadcasted_iota(jnp.int32, (bq, t), 0)
    col = jax.lax.broadcasted_iota(jnp.int32, (bq, t), 1)
    s = jnp.where(col <= row, s, -jnp.inf)
    m = jnp.max(s, axis=1, keepdims=True)
    p = jnp.exp(s - m)
    l = jnp.sum(p, axis=1, keepdims=True)
    colsum_ref[...] += jnp.sum(p / l, axis=0, keepdims=True)

    @pl.when(qi == pl.num_programs(2) - 1)
    def _():
        v = colsum_ref[...] * (1.0 / t)
        cidx = jax.lax.broadcasted_iota(jnp.int32, (1, t), 1)
        v = jnp.where(cidx >= t - recency, 1.0, v)
        imp_ref[0, :, :] = v
        vb = jax.lax.bitcast_convert_type(v, jnp.int32)

        def body(i, tb):
            cand = tb | (1 << (30 - i))
            cnt = jnp.sum((vb >= cand).astype(jnp.int32))
            return jnp.where(cnt >= k_keep, cand, tb)

        tbits = jax.lax.fori_loop(0, 31, body, jnp.int32(0))
        thr_ref[0, :, :] = jnp.broadcast_to(
            jax.lax.bitcast_convert_type(tbits, jnp.float32), (1, 128))


def _sc_body(imp_hbm, thr_hbm, kv_hbm, kk_hbm, vv_hbm,
             imp_v, thr_v, idxk_v, idxv_v, kbuf, vbuf, sem,
             *, b, t, n_head, k_keep, n_cores):
    wid = lax.axis_index("s") * n_cores + lax.axis_index("c")
    bi = wid // n_head
    h = wid % n_head
    base_k = (n_head + h) * (b * t) + bi * t
    base_v = (2 * n_head + h) * (b * t) + bi * t

    pltpu.sync_copy(imp_hbm.at[wid], imp_v)
    pltpu.sync_copy(thr_hbm.at[wid], thr_v)
    thr = thr_v[pl.ds(0, 16)]

    nchunk = t // 16
    iota = lax.iota(jnp.int32, 16)

    def chunk_body(i, cursor):
        x = imp_v[pl.ds(i * 16, 16)]
        msk = x >= thr
        mi = msk.astype(jnp.int32)
        pos = cursor + plsc.cumsum(mi) - 1
        wr = jnp.logical_and(msk, pos < k_keep)
        tvec = i * 16 + iota
        plsc.store_scatter(idxk_v, [pos], tvec + base_k, mask=wr)
        plsc.store_scatter(idxv_v, [pos], tvec + base_v, mask=wr)
        return cursor + jnp.sum(mi)

    lax.fori_loop(0, nchunk, chunk_body, jnp.int32(0))

    pltpu.async_copy(kv_hbm.at[idxk_v], kbuf, sem).wait()
    pltpu.async_copy(kv_hbm.at[idxv_v], vbuf, sem).wait()
    pltpu.sync_copy(kbuf, kk_hbm.at[wid])
    pltpu.sync_copy(vbuf, vv_hbm.at[wid])


def _sc_select_gather(imp, thr, kv32, b, t, n_head, k_keep, dw):
    bh = b * n_head
    info = plsc.get_sparse_core_info()
    mesh = plsc.VectorSubcoreMesh(core_axis_name="c", subcore_axis_name="s")
    fn = pl.kernel(
        functools.partial(_sc_body, b=b, t=t, n_head=n_head, k_keep=k_keep,
                          n_cores=info.num_cores),
        out_type=(jax.ShapeDtypeStruct((bh, k_keep, dw), jnp.int32),
                  jax.ShapeDtypeStruct((bh, k_keep, dw), jnp.int32)),
        mesh=mesh,
        compiler_params=pltpu.CompilerParams(
            use_tc_tiling_on_sc=False, needs_layout_passes=False),
        scratch_types=[
            pltpu.VMEM((t,), jnp.float32),
            pltpu.VMEM((128,), jnp.float32),
            pltpu.VMEM((k_keep,), jnp.int32),
            pltpu.VMEM((k_keep,), jnp.int32),
            pltpu.VMEM((k_keep, dw), jnp.int32),
            pltpu.VMEM((k_keep, dw), jnp.int32),
            pltpu.SemaphoreType.DMA,
        ],
    )
    return fn(imp, thr, kv32)


def _pruned_attn_kernel(q_ref, k_ref, v_ref, o_ref, *, scale):
    q = q_ref[0, 0, :, :]
    k = k_ref[0, :, :]
    v = v_ref[0, :, :]
    s = jax.lax.dot_general(
        q, k, (((1,), (1,)), ((), ())),
        preferred_element_type=jnp.float32) * scale
    m = jnp.max(s, axis=1, keepdims=True)
    p = jnp.exp(s - m)
    l = jnp.sum(p, axis=1, keepdims=True)
    o = jax.lax.dot_general(
        p.astype(jnp.bfloat16), v, (((1,), (0,)), ((), ())),
        preferred_element_type=jnp.float32) / l
    o_ref[0, 0, :, :] = o.astype(jnp.bfloat16)


def _proj_kernel(x_ref, w_ref, o_ref, acc_ref):
    h = pl.program_id(2)

    @pl.when(h == 0)
    def _():
        acc_ref[...] = jnp.zeros_like(acc_ref)

    acc_ref[...] += jax.lax.dot_general(
        x_ref[0, 0, :, :], w_ref[0, :, :],
        (((1,), (0,)), ((), ())),
        preferred_element_type=jnp.float32)

    @pl.when(h == pl.num_programs(2) - 1)
    def _():
        o_ref[0, :, :] = acc_ref[...]


def _forward(x, w_attn, w_proj, n_head, k_keep, recency, bq,
             interpret=False):
    b, t, c = x.shape
    dh = c // n_head
    dw = dh // 2
    scale = np.float32(1.0 / np.sqrt(dh))
    nq = t // bq
    bm = bq
    nm = t // bm

    x2 = x.reshape(b * t, c).astype(jnp.bfloat16)
    w_attn = w_attn.astype(jnp.bfloat16)
    bn = int(np.gcd(8 * dh, 3 * c))
    hpb = bn // dh
    qkv = pl.pallas_call(
        functools.partial(_qkv_kernel, hpb=hpb, dh=dh),
        grid=(b * t // bm, 3 * c // bn),
        in_specs=[
            pl.BlockSpec((bm, c), lambda i, j: (i, 0)),
            pl.BlockSpec((c, bn), lambda i, j: (0, j)),
        ],
        out_specs=pl.BlockSpec(
            (hpb, 1, bm, dh),
            lambda i, j, _nm=nm: (j, i // _nm, i % _nm, 0)),
        out_shape=jax.ShapeDtypeStruct((3 * n_head, b, t, dh), jnp.bfloat16),
        interpret=interpret,
    )(x2, w_attn)

    grid = (b, n_head, nq)
    q_spec = pl.BlockSpec((1, 1, bq, dh), lambda bi, h, qi: (h, bi, qi, 0))
    k_spec = pl.BlockSpec((1, 1, t, dh),
                          lambda bi, h, qi: (h + n_head, bi, 0, 0))

    imp, thr = pl.pallas_call(
        functools.partial(_importance_kernel, t=t, bq=bq, scale=scale,
                          k_keep=k_keep, recency=recency),
        grid=grid,
        in_specs=[q_spec, k_spec],
        out_specs=[
            pl.BlockSpec((1, 1, t), lambda bi, h, qi: (bi * n_head + h, 0, 0)),
            pl.BlockSpec((1, 1, 128),
                         lambda bi, h, qi: (bi * n_head + h, 0, 0)),
        ],
        out_shape=[
            jax.ShapeDtypeStruct((b * n_head, 1, t), jnp.float32),
            jax.ShapeDtypeStruct((b * n_head, 1, 128), jnp.float32),
        ],
        scratch_shapes=[pltpu.VMEM((1, t), jnp.float32)],
        interpret=interpret,
    )(qkv, qkv)

    kv32 = jax.lax.bitcast_convert_type(
        qkv.reshape(3 * n_head * b * t, dw, 2), jnp.int32)
    kk32, vv32 = _sc_select_gather(
        imp.reshape(b * n_head, t), thr.reshape(b * n_head, 128), kv32,
        b, t, n_head, k_keep, dw)
    kk = jax.lax.bitcast_convert_type(kk32, jnp.bfloat16).reshape(
        b * n_head, k_keep, dh)
    vv = jax.lax.bitcast_convert_type(vv32, jnp.bfloat16).reshape(
        b * n_head, k_keep, dh)

    kk_spec = pl.BlockSpec((1, k_keep, dh),
                           lambda bi, h, qi: (bi * n_head + h, 0, 0))
    out_heads = pl.pallas_call(
        functools.partial(_pruned_attn_kernel, scale=scale),
        grid=grid,
        in_specs=[q_spec, kk_spec, kk_spec],
        out_specs=pl.BlockSpec((1, 1, bq, dh),
                               lambda bi, h, qi: (h, bi, qi, 0)),
        out_shape=jax.ShapeDtypeStruct((n_head, b, t, dh), jnp.bfloat16),
        interpret=interpret,
    )(qkv, kk, vv)

    w3 = w_proj.reshape(n_head, dh, c).astype(jnp.bfloat16)
    out = pl.pallas_call(
        _proj_kernel,
        grid=(b, nm, n_head),
        in_specs=[
            pl.BlockSpec((1, 1, bm, dh), lambda bi, mi, h: (h, bi, mi, 0)),
            pl.BlockSpec((1, dh, c), lambda bi, mi, h: (h, 0, 0)),
        ],
        out_specs=pl.BlockSpec((1, bm, c), lambda bi, mi, h: (bi, mi, 0)),
        out_shape=jax.ShapeDtypeStruct((b, t, c), jnp.float32),
        scratch_shapes=[pltpu.VMEM((bm, c), jnp.float32)],
        interpret=interpret,
    )(out_heads, w3)
    return out


def kernel(x, W_attn, W_proj):
    return _forward(x, W_attn, W_proj, n_head=N_HEAD, k_keep=K_KEEP,
                    recency=RECENCY, bq=256)


# speedup vs baseline: 1.0368x; 1.0368x over previous
import functools

import jax
import jax.numpy as jnp
import numpy as np
from jax import lax
from jax.experimental import pallas as pl
from jax.experimental.pallas import tpu as pltpu
from jax.experimental.pallas import tpu_sc as plsc

N_HEAD = 16
K_KEEP = 512
RECENCY = 64


def _qkv_kernel(x_ref, w_ref, o_ref, *, hpb, dh):
    r = jax.lax.dot_general(
        x_ref[...], w_ref[...],
        (((1,), (0,)), ((), ())),
        preferred_element_type=jnp.float32)
    r = r.astype(jnp.bfloat16)
    for ih in range(hpb):
        o_ref[ih, 0, :, :] = r[:, ih * dh:(ih + 1) * dh]


def _importance_kernel(q_ref, k_ref, imp_ref, thr_ref, colsum_ref, ps_ref, *,
                       t, bq, scale, k_keep, recency):
    qi = pl.program_id(2)

    @pl.when(qi == 0)
    def _():
        colsum_ref[...] = jnp.zeros_like(colsum_ref)

    q = q_ref[0, 0, :, :]
    nk = qi + 1
    tri = (jax.lax.broadcasted_iota(jnp.int32, (bq, bq), 0) >=
           jax.lax.broadcasted_iota(jnp.int32, (bq, bq), 1))

    def pass1(j, m_run):
        kj = k_ref[0, 0, pl.ds(j * bq, bq), :]
        sj = jax.lax.dot_general(
            q, kj, (((1,), (1,)), ((), ())),
            preferred_element_type=jnp.float32) * scale
        sj = jnp.where(jnp.logical_or(j < qi, tri), sj, -jnp.inf)
        ps_ref[:, pl.ds(j * bq, bq)] = sj
        return jnp.maximum(m_run, jnp.max(sj, axis=1, keepdims=True))

    m = jax.lax.fori_loop(
        0, nk, pass1, jnp.full((bq, 1), -jnp.inf, jnp.float32))

    def pass2(j, l_run):
        p = jnp.exp(ps_ref[:, pl.ds(j * bq, bq)] - m)
        ps_ref[:, pl.ds(j * bq, bq)] = p
        return l_run + jnp.sum(p, axis=1, keepdims=True)

    l = jax.lax.fori_loop(0, nk, pass2, jnp.zeros((bq, 1), jnp.float32))
    r = 1.0 / l

    def pass3(j, carry):
        p = ps_ref[:, pl.ds(j * bq, bq)]
        colsum_ref[:, pl.ds(j * bq, bq)] += jnp.sum(
            p * r, axis=0, keepdims=True)
        return carry

    jax.lax.fori_loop(0, nk, pass3, jnp.int32(0))

    @pl.when(qi == pl.num_programs(2) - 1)
    def _():
        v = colsum_ref[...] * (1.0 / t)
        cidx = jax.lax.broadcasted_iota(jnp.int32, (1, t), 1)
        v = jnp.where(cidx >= t - recency, 1.0, v)
        imp_ref[0, :, :] = v
        vb = jax.lax.bitcast_convert_type(v, jnp.int32)

        def body(i, tb):
            cand = tb | (1 << (30 - i))
            cnt = jnp.sum((vb >= cand).astype(jnp.int32))
            return jnp.where(cnt >= k_keep, cand, tb)

        tbits = jax.lax.fori_loop(0, 31, body, jnp.int32(0))
        thr_ref[0, :, :] = jnp.broadcast_to(
            jax.lax.bitcast_convert_type(tbits, jnp.float32), (1, 128))


def _sc_body(imp_hbm, thr_hbm, kv_hbm, kk_hbm, vv_hbm,
             imp_v, thr_v, idxk_v, idxv_v, kbuf, vbuf, sem,
             *, b, t, n_head, k_keep, n_cores):
    wid = lax.axis_index("s") * n_cores + lax.axis_index("c")
    bi = wid // n_head
    h = wid % n_head
    base_k = (n_head + h) * (b * t) + bi * t
    base_v = (2 * n_head + h) * (b * t) + bi * t

    pltpu.sync_copy(imp_hbm.at[wid], imp_v)
    pltpu.sync_copy(thr_hbm.at[wid], thr_v)
    thr = thr_v[pl.ds(0, 16)]

    nchunk = t // 16
    iota = lax.iota(jnp.int32, 16)

    def chunk_body(i, cursor):
        x = imp_v[pl.ds(i * 16, 16)]
        msk = x >= thr
        mi = msk.astype(jnp.int32)
        pos = cursor + plsc.cumsum(mi) - 1
        wr = jnp.logical_and(msk, pos < k_keep)
        tvec = i * 16 + iota
        plsc.store_scatter(idxk_v, [pos], tvec + base_k, mask=wr)
        plsc.store_scatter(idxv_v, [pos], tvec + base_v, mask=wr)
        return cursor + jnp.sum(mi)

    lax.fori_loop(0, nchunk, chunk_body, jnp.int32(0))

    pltpu.async_copy(kv_hbm.at[idxk_v], kbuf, sem).wait()
    pltpu.async_copy(kv_hbm.at[idxv_v], vbuf, sem).wait()
    pltpu.sync_copy(kbuf, kk_hbm.at[wid])
    pltpu.sync_copy(vbuf, vv_hbm.at[wid])


def _sc_select_gather(imp, thr, kv, b, t, n_head, k_keep, dw):
    bh = b * n_head
    info = plsc.get_sparse_core_info()
    mesh = plsc.VectorSubcoreMesh(core_axis_name="c", subcore_axis_name="s")
    fn = pl.kernel(
        functools.partial(_sc_body, b=b, t=t, n_head=n_head, k_keep=k_keep,
                          n_cores=info.num_cores),
        out_type=(jax.ShapeDtypeStruct((bh, k_keep, dw), jnp.bfloat16),
                  jax.ShapeDtypeStruct((bh, k_keep, dw), jnp.bfloat16)),
        mesh=mesh,
        compiler_params=pltpu.CompilerParams(
            use_tc_tiling_on_sc=False, needs_layout_passes=False),
        scratch_types=[
            pltpu.VMEM((t,), jnp.float32),
            pltpu.VMEM((128,), jnp.float32),
            pltpu.VMEM((k_keep,), jnp.int32),
            pltpu.VMEM((k_keep,), jnp.int32),
            pltpu.VMEM((k_keep, dw), jnp.bfloat16),
            pltpu.VMEM((k_keep, dw), jnp.bfloat16),
            pltpu.SemaphoreType.DMA,
        ],
    )
    return fn(imp, thr, kv)


def _pruned_attn_kernel(q_ref, k_ref, v_ref, o_ref, *, scale):
    q = q_ref[0, 0, :, :]
    k = k_ref[0, :, :]
    v = v_ref[0, :, :]
    s = jax.lax.dot_general(
        q, k, (((1,), (1,)), ((), ())),
        preferred_element_type=jnp.float32) * scale
    m = jnp.max(s, axis=1, keepdims=True)
    p = jnp.exp(s - m)
    l = jnp.sum(p, axis=1, keepdims=True)
    o = jax.lax.dot_general(
        p.astype(jnp.bfloat16), v, (((1,), (0,)), ((), ())),
        preferred_element_type=jnp.float32) / l
    o_ref[0, 0, :, :] = o.astype(jnp.bfloat16)


def _proj_kernel(x_ref, w_ref, o_ref, acc_ref):
    h = pl.program_id(2)

    @pl.when(h == 0)
    def _():
        acc_ref[...] = jnp.zeros_like(acc_ref)

    acc_ref[...] += jax.lax.dot_general(
        x_ref[0, 0, :, :], w_ref[0, :, :],
        (((1,), (0,)), ((), ())),
        preferred_element_type=jnp.float32)

    @pl.when(h == pl.num_programs(2) - 1)
    def _():
        o_ref[0, :, :] = acc_ref[...]


def _forward(x, w_attn, w_proj, n_head, k_keep, recency, bq,
             interpret=False):
    b, t, c = x.shape
    dh = c // n_head
    scale = np.float32(1.0 / np.sqrt(dh))
    nq = t // bq
    bm = bq
    nm = t // bm

    x2 = x.reshape(b * t, c).astype(jnp.bfloat16)
    w_attn = w_attn.astype(jnp.bfloat16)
    bn = int(np.gcd(8 * dh, 3 * c))
    hpb = bn // dh
    qkv = pl.pallas_call(
        functools.partial(_qkv_kernel, hpb=hpb, dh=dh),
        grid=(b * t // bm, 3 * c // bn),
        in_specs=[
            pl.BlockSpec((bm, c), lambda i, j: (i, 0)),
            pl.BlockSpec((c, bn), lambda i, j: (0, j)),
        ],
        out_specs=pl.BlockSpec(
            (hpb, 1, bm, dh),
            lambda i, j, _nm=nm: (j, i // _nm, i % _nm, 0)),
        out_shape=jax.ShapeDtypeStruct((3 * n_head, b, t, dh), jnp.bfloat16),
        interpret=interpret,
    )(x2, w_attn)

    grid = (b, n_head, nq)
    q_spec = pl.BlockSpec((1, 1, bq, dh), lambda bi, h, qi: (h, bi, qi, 0))
    k_spec = pl.BlockSpec((1, 1, t, dh),
                          lambda bi, h, qi: (h + n_head, bi, 0, 0))

    imp, thr = pl.pallas_call(
        functools.partial(_importance_kernel, t=t, bq=bq, scale=scale,
                          k_keep=k_keep, recency=recency),
        grid=grid,
        in_specs=[q_spec, k_spec],
        out_specs=[
            pl.BlockSpec((1, 1, t), lambda bi, h, qi: (bi * n_head + h, 0, 0)),
            pl.BlockSpec((1, 1, 128),
                         lambda bi, h, qi: (bi * n_head + h, 0, 0)),
        ],
        out_shape=[
            jax.ShapeDtypeStruct((b * n_head, 1, t), jnp.float32),
            jax.ShapeDtypeStruct((b * n_head, 1, 128), jnp.float32),
        ],
        scratch_shapes=[pltpu.VMEM((1, t), jnp.float32),
                        pltpu.VMEM((bq, t), jnp.float32)],
        interpret=interpret,
    )(qkv, qkv)

    kk, vv = _sc_select_gather(
        imp.reshape(b * n_head, t), thr.reshape(b * n_head, 128),
        qkv.reshape(3 * n_head * b * t, dh),
        b, t, n_head, k_keep, dh)

    kk_spec = pl.BlockSpec((1, k_keep, dh),
                           lambda bi, h, qi: (bi * n_head + h, 0, 0))
    out_heads = pl.pallas_call(
        functools.partial(_pruned_attn_kernel, scale=scale),
        grid=grid,
        in_specs=[q_spec, kk_spec, kk_spec],
        out_specs=pl.BlockSpec((1, 1, bq, dh),
                               lambda bi, h, qi: (h, bi, qi, 0)),
        out_shape=jax.ShapeDtypeStruct((n_head, b, t, dh), jnp.bfloat16),
        interpret=interpret,
    )(qkv, kk, vv)

    w3 = w_proj.reshape(n_head, dh, c).astype(jnp.bfloat16)
    out = pl.pallas_call(
        _proj_kernel,
        grid=(b, nm, n_head),
        in_specs=[
            pl.BlockSpec((1, 1, bm, dh), lambda bi, mi, h: (h, bi, mi, 0)),
            pl.BlockSpec((1, dh, c), lambda bi, mi, h: (h, 0, 0)),
        ],
        out_specs=pl.BlockSpec((1, bm, c), lambda bi, mi, h: (bi, mi, 0)),
        out_shape=jax.ShapeDtypeStruct((b, t, c), jnp.float32),
        scratch_shapes=[pltpu.VMEM((bm, c), jnp.float32)],
        interpret=interpret,
    )(out_heads, w3)
    return out


def kernel(x, W_attn, W_proj):
    return _forward(x, W_attn, W_proj, n_head=N_HEAD, k_keep=K_KEEP,
                    recency=RECENCY, bq=256)


# speedup vs baseline: 1.0372x; 1.0004x over previous
import functools

import jax
import jax.numpy as jnp
import numpy as np
from jax import lax
from jax.experimental import pallas as pl
from jax.experimental.pallas import tpu as pltpu
from jax.experimental.pallas import tpu_sc as plsc

N_HEAD = 16
K_KEEP = 512
RECENCY = 64


def _qkv_kernel(x_ref, w_ref, o_ref, *, hpb, dh):
    r = jax.lax.dot_general(
        x_ref[...], w_ref[...],
        (((1,), (0,)), ((), ())),
        preferred_element_type=jnp.float32)
    r = r.astype(jnp.bfloat16)
    for ih in range(hpb):
        o_ref[ih, 0, :, :] = r[:, ih * dh:(ih + 1) * dh]


def _importance_kernel(q_ref, k_ref, imp_ref, thr_ref, colsum_ref, ps_ref, *,
                       t, bq, scale, k_keep, recency):
    qi = pl.program_id(2)

    @pl.when(qi == 0)
    def _():
        colsum_ref[...] = jnp.zeros_like(colsum_ref)

    q = q_ref[0, 0, :, :]
    nk = qi + 1
    tri = (jax.lax.broadcasted_iota(jnp.int32, (bq, bq), 0) >=
           jax.lax.broadcasted_iota(jnp.int32, (bq, bq), 1))

    def pass1(j, m_run):
        kj = k_ref[0, 0, pl.ds(j * bq, bq), :]
        sj = jax.lax.dot_general(
            q, kj, (((1,), (1,)), ((), ())),
            preferred_element_type=jnp.float32) * scale
        sj = jnp.where(jnp.logical_or(j < qi, tri), sj, -jnp.inf)
        ps_ref[:, pl.ds(j * bq, bq)] = sj
        return jnp.maximum(m_run, jnp.max(sj, axis=1, keepdims=True))

    m = jax.lax.fori_loop(
        0, nk, pass1, jnp.full((bq, 1), -jnp.inf, jnp.float32))

    def pass2(j, l_run):
        p = jnp.exp(ps_ref[:, pl.ds(j * bq, bq)] - m)
        ps_ref[:, pl.ds(j * bq, bq)] = p
        return l_run + jnp.sum(p, axis=1, keepdims=True)

    l = jax.lax.fori_loop(0, nk, pass2, jnp.zeros((bq, 1), jnp.float32))
    r = 1.0 / l

    def pass3(j, carry):
        p = ps_ref[:, pl.ds(j * bq, bq)]
        colsum_ref[:, pl.ds(j * bq, bq)] += jnp.sum(
            p * r, axis=0, keepdims=True)
        return carry

    jax.lax.fori_loop(0, nk, pass3, jnp.int32(0))

    @pl.when(qi == pl.num_programs(2) - 1)
    def _():
        v = colsum_ref[...] * (1.0 / t)
        cidx = jax.lax.broadcasted_iota(jnp.int32, (1, t), 1)
        v = jnp.where(cidx >= t - recency, 1.0, v)
        imp_ref[0, :, :] = v
        vb = jax.lax.bitcast_convert_type(v, jnp.int32)

        def body(i, tb):
            cand = tb | (1 << (30 - i))
            cnt = jnp.sum((vb >= cand).astype(jnp.int32))
            return jnp.where(cnt >= k_keep, cand, tb)

        tbits = jax.lax.fori_loop(0, 31, body, jnp.int32(0))
        thr_ref[0, :, :] = jnp.broadcast_to(
            jax.lax.bitcast_convert_type(tbits, jnp.float32), (1, 128))


def _sc_body(imp_hbm, thr_hbm, kv_hbm, kk_hbm, vv_hbm,
             imp_v, thr_v, idxk_v, idxv_v, kbuf, vbuf, sem,
             *, b, t, n_head, k_keep, n_cores):
    wid = lax.axis_index("s") * n_cores + lax.axis_index("c")
    bi = wid // n_head
    h = wid % n_head
    base_k = (n_head + h) * (b * t) + bi * t
    base_v = (2 * n_head + h) * (b * t) + bi * t

    pltpu.sync_copy(imp_hbm.at[wid], imp_v)
    pltpu.sync_copy(thr_hbm.at[wid], thr_v)
    thr = thr_v[pl.ds(0, 16)]

    nchunk = t // 16
    iota = lax.iota(jnp.int32, 16)

    @plsc.parallel_loop(0, nchunk, step=1, unroll=8, carry=jnp.int32(0))
    def _loop(i, cursor):
        x = imp_v[pl.ds(i * 16, 16)]
        msk = x >= thr
        mi = msk.astype(jnp.int32)
        pos = cursor + plsc.cumsum(mi) - 1
        wr = jnp.logical_and(msk, pos < k_keep)
        tvec = i * 16 + iota
        plsc.store_scatter(idxk_v, [pos], tvec + base_k, mask=wr)
        plsc.store_scatter(idxv_v, [pos], tvec + base_v, mask=wr)
        return cursor + jnp.sum(mi)

    pltpu.async_copy(kv_hbm.at[idxk_v], kbuf, sem).wait()
    pltpu.async_copy(kv_hbm.at[idxv_v], vbuf, sem).wait()
    pltpu.sync_copy(kbuf, kk_hbm.at[wid])
    pltpu.sync_copy(vbuf, vv_hbm.at[wid])


def _sc_select_gather(imp, thr, kv, b, t, n_head, k_keep, dw):
    bh = b * n_head
    info = plsc.get_sparse_core_info()
    mesh = plsc.VectorSubcoreMesh(core_axis_name="c", subcore_axis_name="s")
    fn = pl.kernel(
        functools.partial(_sc_body, b=b, t=t, n_head=n_head, k_keep=k_keep,
                          n_cores=info.num_cores),
        out_type=(jax.ShapeDtypeStruct((bh, k_keep, dw), jnp.bfloat16),
                  jax.ShapeDtypeStruct((bh, k_keep, dw), jnp.bfloat16)),
        mesh=mesh,
        compiler_params=pltpu.CompilerParams(
            use_tc_tiling_on_sc=False, needs_layout_passes=False),
        scratch_types=[
            pltpu.VMEM((t,), jnp.float32),
            pltpu.VMEM((128,), jnp.float32),
            pltpu.VMEM((k_keep,), jnp.int32),
            pltpu.VMEM((k_keep,), jnp.int32),
            pltpu.VMEM((k_keep, dw), jnp.bfloat16),
            pltpu.VMEM((k_keep, dw), jnp.bfloat16),
            pltpu.SemaphoreType.DMA,
        ],
    )
    return fn(imp, thr, kv)


def _pruned_attn_kernel(q_ref, k_ref, v_ref, o_ref, *, scale):
    q = q_ref[0, 0, :, :]
    k = k_ref[0, :, :]
    v = v_ref[0, :, :]
    s = jax.lax.dot_general(
        q, k, (((1,), (1,)), ((), ())),
        preferred_element_type=jnp.float32) * scale
    m = jnp.max(s, axis=1, keepdims=True)
    p = jnp.exp(s - m)
    l = jnp.sum(p, axis=1, keepdims=True)
    o = jax.lax.dot_general(
        p.astype(jnp.bfloat16), v, (((1,), (0,)), ((), ())),
        preferred_element_type=jnp.float32) / l
    o_ref[0, 0, :, :] = o.astype(jnp.bfloat16)


def _proj_kernel(x_ref, w_ref, o_ref, acc_ref):
    h = pl.program_id(2)

    @pl.when(h == 0)
    def _():
        acc_ref[...] = jnp.zeros_like(acc_ref)

    acc_ref[...] += jax.lax.dot_general(
        x_ref[0, 0, :, :], w_ref[0, :, :],
        (((1,), (0,)), ((), ())),
        preferred_element_type=jnp.float32)

    @pl.when(h == pl.num_programs(2) - 1)
    def _():
        o_ref[0, :, :] = acc_ref[...]


def _forward(x, w_attn, w_proj, n_head, k_keep, recency, bq,
             interpret=False):
    b, t, c = x.shape
    dh = c // n_head
    scale = np.float32(1.0 / np.sqrt(dh))
    nq = t // bq
    bm = bq
    nm = t // bm

    x2 = x.reshape(b * t, c).astype(jnp.bfloat16)
    w_attn = w_attn.astype(jnp.bfloat16)
    bn = int(np.gcd(8 * dh, 3 * c))
    hpb = bn // dh
    qkv = pl.pallas_call(
        functools.partial(_qkv_kernel, hpb=hpb, dh=dh),
        grid=(b * t // bm, 3 * c // bn),
        in_specs=[
            pl.BlockSpec((bm, c), lambda i, j: (i, 0)),
            pl.BlockSpec((c, bn), lambda i, j: (0, j)),
        ],
        out_specs=pl.BlockSpec(
            (hpb, 1, bm, dh),
            lambda i, j, _nm=nm: (j, i // _nm, i % _nm, 0)),
        out_shape=jax.ShapeDtypeStruct((3 * n_head, b, t, dh), jnp.bfloat16),
        interpret=interpret,
    )(x2, w_attn)

    grid = (b, n_head, nq)
    q_spec = pl.BlockSpec((1, 1, bq, dh), lambda bi, h, qi: (h, bi, qi, 0))
    k_spec = pl.BlockSpec((1, 1, t, dh),
                          lambda bi, h, qi: (h + n_head, bi, 0, 0))

    imp, thr = pl.pallas_call(
        functools.partial(_importance_kernel, t=t, bq=bq, scale=scale,
                          k_keep=k_keep, recency=recency),
        grid=grid,
        in_specs=[q_spec, k_spec],
        out_specs=[
            pl.BlockSpec((1, 1, t), lambda bi, h, qi: (bi * n_head + h, 0, 0)),
            pl.BlockSpec((1, 1, 128),
                         lambda bi, h, qi: (bi * n_head + h, 0, 0)),
        ],
        out_shape=[
            jax.ShapeDtypeStruct((b * n_head, 1, t), jnp.float32),
            jax.ShapeDtypeStruct((b * n_head, 1, 128), jnp.float32),
        ],
        scratch_shapes=[pltpu.VMEM((1, t), jnp.float32),
                        pltpu.VMEM((bq, t), jnp.float32)],
        interpret=interpret,
    )(qkv, qkv)

    kk, vv = _sc_select_gather(
        imp.reshape(b * n_head, t), thr.reshape(b * n_head, 128),
        qkv.reshape(3 * n_head * b * t, dh),
        b, t, n_head, k_keep, dh)

    kk_spec = pl.BlockSpec((1, k_keep, dh),
                           lambda bi, h, qi: (bi * n_head + h, 0, 0))
    out_heads = pl.pallas_call(
        functools.partial(_pruned_attn_kernel, scale=scale),
        grid=grid,
        in_specs=[q_spec, kk_spec, kk_spec],
        out_specs=pl.BlockSpec((1, 1, bq, dh),
                               lambda bi, h, qi: (h, bi, qi, 0)),
        out_shape=jax.ShapeDtypeStruct((n_head, b, t, dh), jnp.bfloat16),
        interpret=interpret,
    )(qkv, kk, vv)

    w3 = w_proj.reshape(n_head, dh, c).astype(jnp.bfloat16)
    out = pl.pallas_call(
        _proj_kernel,
        grid=(b, nm, n_head),
        in_specs=[
            pl.BlockSpec((1, 1, bm, dh), lambda bi, mi, h: (h, bi, mi, 0)),
            pl.BlockSpec((1, dh, c), lambda bi, mi, h: (h, 0, 0)),
        ],
        out_specs=pl.BlockSpec((1, bm, c), lambda bi, mi, h: (bi, mi, 0)),
        out_shape=jax.ShapeDtypeStruct((b, t, c), jnp.float32),
        scratch_shapes=[pltpu.VMEM((bm, c), jnp.float32)],
        interpret=interpret,
    )(out_heads, w3)
    return out


def kernel(x, W_attn, W_proj):
    return _forward(x, W_attn, W_proj, n_head=N_HEAD, k_keep=K_KEEP,
                    recency=RECENCY, bq=256)


# speedup vs baseline: 1.3305x; 1.2828x over previous
import functools

import jax
import jax.numpy as jnp
import numpy as np
from jax import lax
from jax.experimental import pallas as pl
from jax.experimental.pallas import tpu as pltpu
from jax.experimental.pallas import tpu_sc as plsc

N_HEAD = 16
K_KEEP = 512
RECENCY = 64


def _qkv_kernel(x_ref, w_ref, o_ref, *, hpb, dh):
    r = jax.lax.dot_general(
        x_ref[...], w_ref[...],
        (((1,), (0,)), ((), ())),
        preferred_element_type=jnp.float32)
    r = r.astype(jnp.bfloat16)
    for ih in range(hpb):
        o_ref[ih, 0, :, :] = r[:, ih * dh:(ih + 1) * dh]


def _importance_kernel(q_ref, k_ref, imp_ref, thr_ref, colsum_ref, ps_ref, *,
                       t, bq, scale, k_keep, recency):
    qi = pl.program_id(2)

    @pl.when(qi == 0)
    def _():
        colsum_ref[...] = jnp.zeros_like(colsum_ref)

    q = q_ref[0, 0, :, :]
    k = k_ref[0, 0, :, :]
    s = jax.lax.dot_general(
        q, k, (((1,), (1,)), ((), ())),
        preferred_element_type=jnp.float32) * scale
    row = qi * bq + jax.lax.broadcasted_iota(jnp.int32, (bq, t), 0)
    col = jax.lax.broadcasted_iota(jnp.int32, (bq, t), 1)
    s = jnp.where(col <= row, s, -jnp.inf)
    m = jnp.max(s, axis=1, keepdims=True)
    p = jnp.exp(s - m)
    l = jnp.sum(p, axis=1, keepdims=True)
    colsum_ref[...] += jnp.sum(p / l, axis=0, keepdims=True)

    @pl.when(qi == pl.num_programs(2) - 1)
    def _():
        v = colsum_ref[...] * (1.0 / t)
        cidx = jax.lax.broadcasted_iota(jnp.int32, (1, t), 1)
        v = jnp.where(cidx >= t - recency, 1.0, v)
        imp_ref[0, :, :] = v
        vb = jax.lax.bitcast_convert_type(v, jnp.int32)

        def body(i, tb):
            cand = tb | (1 << (30 - i))
            cnt = jnp.sum((vb >= cand).astype(jnp.int32))
            return jnp.where(cnt >= k_keep, cand, tb)

        tbits = jax.lax.fori_loop(0, 31, body, jnp.int32(0))
        thr_ref[0, :, :] = jnp.broadcast_to(
            jax.lax.bitcast_convert_type(tbits, jnp.float32), (1, 128))


def _sc_body(imp_hbm, thr_hbm, kv_hbm, kk_hbm, vv_hbm,
             imp_v, thr_v, idxk_v, idxv_v, kbuf, vbuf, sem,
             *, b, t, n_head, k_keep, n_cores):
    wid = lax.axis_index("s") * n_cores + lax.axis_index("c")
    bi = wid // n_head
    h = wid % n_head
    base_k = (n_head + h) * (b * t) + bi * t
    base_v = (2 * n_head + h) * (b * t) + bi * t

    pltpu.sync_copy(imp_hbm.at[wid], imp_v)
    pltpu.sync_copy(thr_hbm.at[wid], thr_v)
    thr = thr_v[pl.ds(0, 16)]

    nchunk = t // 16
    iota = lax.iota(jnp.int32, 16)

    @plsc.parallel_loop(0, nchunk, step=1, unroll=8, carry=jnp.int32(0))
    def _loop(i, cursor):
        x = imp_v[pl.ds(i * 16, 16)]
        msk = x >= thr
        mi = msk.astype(jnp.int32)
        pos = cursor + plsc.cumsum(mi) - 1
        wr = jnp.logical_and(msk, pos < k_keep)
        tvec = i * 16 + iota
        plsc.store_scatter(idxk_v, [pos], tvec + base_k, mask=wr)
        plsc.store_scatter(idxv_v, [pos], tvec + base_v, mask=wr)
        return cursor + jnp.sum(mi)

    pltpu.async_copy(kv_hbm.at[idxk_v], kbuf, sem).wait()
    pltpu.async_copy(kv_hbm.at[idxv_v], vbuf, sem).wait()
    pltpu.sync_copy(kbuf, kk_hbm.at[wid])
    pltpu.sync_copy(vbuf, vv_hbm.at[wid])


def _sc_select_gather(imp, thr, kv, b, t, n_head, k_keep, dw):
    bh = b * n_head
    info = plsc.get_sparse_core_info()
    mesh = plsc.VectorSubcoreMesh(core_axis_name="c", subcore_axis_name="s")
    fn = pl.kernel(
        functools.partial(_sc_body, b=b, t=t, n_head=n_head, k_keep=k_keep,
                          n_cores=info.num_cores),
        out_type=(jax.ShapeDtypeStruct((bh, k_keep, dw), jnp.bfloat16),
                  jax.ShapeDtypeStruct((bh, k_keep, dw), jnp.bfloat16)),
        mesh=mesh,
        compiler_params=pltpu.CompilerParams(
            use_tc_tiling_on_sc=False, needs_layout_passes=False),
        scratch_types=[
            pltpu.VMEM((t,), jnp.float32),
            pltpu.VMEM((128,), jnp.float32),
            pltpu.VMEM((k_keep,), jnp.int32),
            pltpu.VMEM((k_keep,), jnp.int32),
            pltpu.VMEM((k_keep, dw), jnp.bfloat16),
            pltpu.VMEM((k_keep, dw), jnp.bfloat16),
            pltpu.SemaphoreType.DMA,
        ],
    )
    return fn(imp, thr, kv)


def _pruned_attn_kernel(q_ref, k_ref, v_ref, o_ref, *, scale):
    q = q_ref[0, 0, :, :]
    k = k_ref[0, :, :]
    v = v_ref[0, :, :]
    s = jax.lax.dot_general(
        q, k, (((1,), (1,)), ((), ())),
        preferred_element_type=jnp.float32) * scale
    m = jnp.max(s, axis=1, keepdims=True)
    p = jnp.exp(s - m)
    l = jnp.sum(p, axis=1, keepdims=True)
    o = jax.lax.dot_general(
        p.astype(jnp.bfloat16), v, (((1,), (0,)), ((), ())),
        preferred_element_type=jnp.float32) / l
    o_ref[0, 0, :, :] = o.astype(jnp.bfloat16)


def _proj_kernel(x_ref, w_ref, o_ref, acc_ref):
    h = pl.program_id(2)

    @pl.when(h == 0)
    def _():
        acc_ref[...] = jnp.zeros_like(acc_ref)

    acc_ref[...] += jax.lax.dot_general(
        x_ref[0, 0, :, :], w_ref[0, :, :],
        (((1,), (0,)), ((), ())),
        preferred_element_type=jnp.float32)

    @pl.when(h == pl.num_programs(2) - 1)
    def _():
        o_ref[0, :, :] = acc_ref[...]


def _forward(x, w_attn, w_proj, n_head, k_keep, recency, bq,
             interpret=False):
    b, t, c = x.shape
    dh = c // n_head
    scale = np.float32(1.0 / np.sqrt(dh))
    nq = t // bq
    bm = bq
    nm = t // bm

    x2 = x.reshape(b * t, c).astype(jnp.bfloat16)
    w_attn = w_attn.astype(jnp.bfloat16)
    bn = int(np.gcd(8 * dh, 3 * c))
    hpb = bn // dh
    qkv = pl.pallas_call(
        functools.partial(_qkv_kernel, hpb=hpb, dh=dh),
        grid=(b * t // bm, 3 * c // bn),
        in_specs=[
            pl.BlockSpec((bm, c), lambda i, j: (i, 0)),
            pl.BlockSpec((c, bn), lambda i, j: (0, j)),
        ],
        out_specs=pl.BlockSpec(
            (hpb, 1, bm, dh),
            lambda i, j, _nm=nm: (j, i // _nm, i % _nm, 0)),
        out_shape=jax.ShapeDtypeStruct((3 * n_head, b, t, dh), jnp.bfloat16),
        interpret=interpret,
    )(x2, w_attn)

    grid = (b, n_head, nq)
    q_spec = pl.BlockSpec((1, 1, bq, dh), lambda bi, h, qi: (h, bi, qi, 0))
    k_spec = pl.BlockSpec((1, 1, t, dh),
                          lambda bi, h, qi: (h + n_head, bi, 0, 0))

    imp, thr = pl.pallas_call(
        functools.partial(_importance_kernel, t=t, bq=bq, scale=scale,
                          k_keep=k_keep, recency=recency),
        grid=grid,
        in_specs=[q_spec, k_spec],
        out_specs=[
            pl.BlockSpec((1, 1, t), lambda bi, h, qi: (bi * n_head + h, 0, 0)),
            pl.BlockSpec((1, 1, 128),
                         lambda bi, h, qi: (bi * n_head + h, 0, 0)),
        ],
        out_shape=[
            jax.ShapeDtypeStruct((b * n_head, 1, t), jnp.float32),
            jax.ShapeDtypeStruct((b * n_head, 1, 128), jnp.float32),
        ],
        scratch_shapes=[pltpu.VMEM((1, t), jnp.float32),
                        pltpu.VMEM((bq, t), jnp.float32)],
        interpret=interpret,
    )(qkv, qkv)

    kk, vv = _sc_select_gather(
        imp.reshape(b * n_head, t), thr.reshape(b * n_head, 128),
        qkv.reshape(3 * n_head * b * t, dh),
        b, t, n_head, k_keep, dh)

    kk_spec = pl.BlockSpec((1, k_keep, dh),
                           lambda bi, h, qi: (bi * n_head + h, 0, 0))
    out_heads = pl.pallas_call(
        functools.partial(_pruned_attn_kernel, scale=scale),
        grid=grid,
        in_specs=[q_spec, kk_spec, kk_spec],
        out_specs=pl.BlockSpec((1, 1, bq, dh),
                               lambda bi, h, qi: (h, bi, qi, 0)),
        out_shape=jax.ShapeDtypeStruct((n_head, b, t, dh), jnp.bfloat16),
        interpret=interpret,
    )(qkv, kk, vv)

    w3 = w_proj.reshape(n_head, dh, c).astype(jnp.bfloat16)
    out = pl.pallas_call(
        _proj_kernel,
        grid=(b, nm, n_head),
        in_specs=[
            pl.BlockSpec((1, 1, bm, dh), lambda bi, mi, h: (h, bi, mi, 0)),
            pl.BlockSpec((1, dh, c), lambda bi, mi, h: (h, 0, 0)),
        ],
        out_specs=pl.BlockSpec((1, bm, c), lambda bi, mi, h: (bi, mi, 0)),
        out_shape=jax.ShapeDtypeStruct((b, t, c), jnp.float32),
        scratch_shapes=[pltpu.VMEM((bm, c), jnp.float32)],
        interpret=interpret,
    )(out_heads, w3)
    return out


def kernel(x, W_attn, W_proj):
    return _forward(x, W_attn, W_proj, n_head=N_HEAD, k_keep=K_KEEP,
                    recency=RECENCY, bq=256)
